# baseline (speedup 1.0000x reference)
import functools

import jax
import jax.numpy as jnp
from jax import lax
from jax.experimental import pallas as pl
from jax.experimental.pallas import tpu as pltpu
from jax.experimental.pallas import tpu_sc as plsc

_N = 100000
_G = 1024
_C = 128
_BS = 32
_NB = _N // _BS
_NW = 32
_TRASH = _G

_BPW = 104
_IDPAD = _BPW * (_NW - 1) * _BS + 112 * _BS

_RPG = 4000
_BPG = _RPG // _BS
_BPGP = 128


def _make_sc_a():
    mesh = plsc.VectorSubcoreMesh(core_axis_name="c", subcore_axis_name="s")

    @functools.partial(
        pl.kernel,
        mesh=mesh,
        out_type=(
            jax.ShapeDtypeStruct((2, _G, _C), jnp.float32),
            jax.ShapeDtypeStruct((_NW, 7, 1, 16), jnp.int32),
        ),
        scratch_types=[
            pltpu.VMEM((3584,), jnp.int32),
            pltpu.VMEM((7, 1, 16), jnp.int32),
            pltpu.VMEM((128,), jnp.int32),
            pltpu.VMEM((128,), jnp.int32),
            pltpu.VMEM((4, _BS, _C), jnp.float32),
            pltpu.VMEM_SHARED((_G + 1, _C), jnp.float32),
        ] + [pltpu.SemaphoreType.DMA] * 8,
    )
    def sc_a(x_hbm, idsf_hbm, zeros_hbm, out_hbm, sidx_hbm,
             ids_f, sidx_v, sidx_f, imp_v, rows_v, accum, *sems):
        cid = lax.axis_index("c")
        sid = lax.axis_index("s")
        wid = sid * 2 + cid
        fsem, ssem = sems[:4], sems[4:]

        b0 = _BPW * wid
        bcnt = jnp.clip(_NB - b0, 0, _BPW)

        pltpu.sync_copy(
            idsf_hbm.at[pl.ds(pl.multiple_of(b0 * _BS, _BS), _BPW * _BS)],
            ids_f.at[pl.ds(0, _BPW * _BS)])

        pltpu.sync_copy(
            zeros_hbm.at[pl.ds(sid * (_G // 16), _G // 16)],
            accum.at[pl.ds(sid * (_G // 16), _G // 16)],
        )

        @pl.when(sid == 0)
        def _():
            pltpu.sync_copy(zeros_hbm.at[pl.ds(0, 1)],
                            accum.at[pl.ds(_G, 1)])

        zvec = jnp.zeros((16,), jnp.int32)

        def cbody(j, m):
            f = ids_f[pl.ds(pl.multiple_of(j * _BS, _BS), 16)][0]
            last = ids_f[pl.ds(pl.multiple_of(j * _BS + 16, 16), 16)][15]
            valid = j < bcnt
            sidx_f[pl.ds(j, 16)] = zvec + jnp.where(
                (f == last) & valid, f, _TRASH)
            imp = ((f != last) & valid).astype(jnp.int32)

            @pl.when(imp == 1)
            def _():
                imp_v[pl.ds(m, 16)] = zvec + j

            return m + imp

        M = lax.fori_loop(0, 112, cbody, wid * 0)

        for t in range(7):
            sidx_v[t, 0, :] = sidx_f[pl.ds(16 * t, 16)]

        pltpu.sync_copy(sidx_v, sidx_hbm.at[wid])
        plsc.subcore_barrier()

        def impure_block(k):
            return imp_v[pl.ds(k, 16)][0]

        def fetch(k, b):
            jg = b0 + impure_block(k)
            return pltpu.async_copy(
                x_hbm.at[pl.ds(pl.multiple_of(jg * _BS, _BS), _BS)],
                rows_v.at[b], fsem[b])

        def wait_fetch(b):
            pltpu.make_async_copy(
                x_hbm.at[pl.ds(0, _BS)], rows_v.at[b], fsem[b]).wait()

        def scat(k, b):
            jl = impure_block(k)
            ia = ids_f[pl.ds(pl.multiple_of(jl * _BS, _BS), 16)]
            ib = ids_f[pl.ds(pl.multiple_of(jl * _BS + 16, 16), 16)]
            pltpu.async_copy(rows_v.at[b, pl.ds(0, 16)], accum.at[ia],
                             ssem[b], add=True)
            pltpu.async_copy(rows_v.at[b, pl.ds(16, 16)], accum.at[ib],
                             ssem[b], add=True)

        def wait_scat(b):
            pltpu.make_async_copy(
                rows_v.at[b], accum.at[pl.ds(0, _BS)], ssem[b]).wait()

        for pb in range(2):
            @pl.when(M > pb)
            def _(pb=pb):
                fetch(pb, pb)

        def ibody(i, carry):
            for b in range(4):
                k = i * 4 + b
                kf = k + 2
                bf = (b + 2) % 4

                @pl.when(kf < M)
                def _():
                    @pl.when(kf >= 4)
                    def _():
                        wait_scat(bf)

                    fetch(kf, bf)

                @pl.when(k < M)
                def _():
                    wait_fetch(b)
                    scat(k, b)

            return carry

        lax.fori_loop(0, (M + 3) // 4, ibody, 0)

        for b in range(4):
            @pl.when(b < jnp.minimum(M, 4))
            def _(b=b):
                wait_scat(b)

        plsc.subcore_barrier()

        pltpu.sync_copy(
            accum.at[pl.ds(sid * (_G // 16), _G // 16)],
            out_hbm.at[cid].at[pl.ds(sid * (_G // 16), _G // 16)],
        )

    return sc_a


def _make_sc_b():
    mesh = plsc.VectorSubcoreMesh(core_axis_name="c", subcore_axis_name="s")

    @functools.partial(
        pl.kernel,
        mesh=mesh,
        out_type=jax.ShapeDtypeStruct((2, _G, _C), jnp.float32),
        scratch_types=[
            pltpu.VMEM((7, 1, 16), jnp.int32),
            pltpu.VMEM((7, 16, _C), jnp.float32),
            pltpu.VMEM_SHARED((_G + 1, _C), jnp.float32),
            pltpu.SemaphoreType.DMA,
        ],
    )
    def sc_b(s_hbm, sidx_hbm, zeros_hbm, out_hbm, sidx_v, srows_v, accum,
             gsem):
        cid = lax.axis_index("c")
        sid = lax.axis_index("s")
        wid = sid * 2 + cid

        pltpu.sync_copy(
            zeros_hbm.at[pl.ds(sid * (_G // 16), _G // 16)],
            accum.at[pl.ds(sid * (_G // 16), _G // 16)],
        )

        @pl.when(sid == 0)
        def _():
            pltpu.sync_copy(zeros_hbm.at[pl.ds(0, 1)],
                            accum.at[pl.ds(_G, 1)])

        pltpu.sync_copy(sidx_hbm.at[wid], sidx_v)

        pltpu.sync_copy(s_hbm.at[wid], srows_v)

        plsc.subcore_barrier()

        for u in range(7):
            pltpu.sync_copy(srows_v.at[u], accum.at[sidx_v.at[u, 0]],
                            add=True)

        plsc.subcore_barrier()

        pltpu.sync_copy(
            accum.at[pl.ds(sid * (_G // 16), _G // 16)],
            out_hbm.at[cid].at[pl.ds(sid * (_G // 16), _G // 16)],
        )

    return sc_b


_sc_cache = {}


def _sc_kernel(name, maker, *args):
    if name not in _sc_cache:
        _sc_cache[name] = maker()
    return _sc_cache[name](*args)


def _blocksum_body(x_ref, s_ref):
    for i in range(_BPG):
        s_ref[0, i:i + 1, :] = jnp.sum(x_ref[i * _BS:(i + 1) * _BS, :],
                                       axis=0, keepdims=True)


def _blocksum(x):
    s = pl.pallas_call(
        _blocksum_body,
        grid=(_N // _RPG,),
        in_specs=[pl.BlockSpec((_RPG, _C), lambda i: (i, 0))],
        out_specs=pl.BlockSpec((1, _BPGP, _C), lambda i: (i, 0, 0)),
        out_shape=jax.ShapeDtypeStruct((_N // _RPG, _BPGP, _C), jnp.float32),
    )(x)
    return s.reshape((_N // _RPG) * _BPGP, _C)


def _dense_body(pa_ref, pb_ref, aux_ref, gam_ref, bet_ref, w1_ref, b1_ref,
                w2_ref, b2_ref, out_ref, gf_ref):
    gf = pa_ref[0] + pa_ref[1] + pb_ref[0] + pb_ref[1]
    gf_ref[...] = gf
    ax = aux_ref[...]

    mg = jnp.mean(gf, axis=0, keepdims=True)
    vg = jnp.mean((gf - mg) ** 2, axis=0, keepdims=True)
    xg = (gf - mg) * lax.rsqrt(vg + 1e-5) * gam_ref[:, :_C] + bet_ref[:, :_C]

    ma = jnp.mean(ax, axis=0, keepdims=True)
    va = jnp.mean((ax - ma) ** 2, axis=0, keepdims=True)
    xa = (ax - ma) * lax.rsqrt(va + 1e-5) * gam_ref[:, _C:] + bet_ref[:, _C:]

    dn = (((1,), (1,)), ((), ()))
    h = lax.dot_general(xg, w1_ref[:, :_C], dn,
                        preferred_element_type=jnp.float32)
    h = h + lax.dot_general(xa, w1_ref[:, _C:], dn,
                            preferred_element_type=jnp.float32)
    h = jnp.maximum(h + b1_ref[...], 0.0)
    out_ref[...] = lax.dot_general(h, w2_ref[...], dn,
                                   preferred_element_type=jnp.float32) + b2_ref[...]


def kernel(raw_node_out, num_graphs, graph_nodes_list, auxiliary_features,
           bn_gamma, bn_beta, W1, b1, W2, b2):
    del num_graphs
    ids_flat = jnp.pad(graph_nodes_list.astype(jnp.int32),
                       (0, _IDPAD - _N))
    zeros = jnp.zeros((_G, _C), jnp.float32)

    partials_a, sidx = _sc_kernel("a", _make_sc_a, raw_node_out, ids_flat,
                                  zeros)
    s_flat = _blocksum(raw_node_out)
    gidx = jnp.arange(_NW * 112, dtype=jnp.int32).reshape(_NW, 112) % 112
    base = (jnp.arange(_NW, dtype=jnp.int32) * _BPW)[:, None]
    g = base + gidx
    fp = jnp.where(g < _NB, (g // _BPG) * _BPGP + g % _BPG, 0)
    sw = s_flat[fp.reshape(-1)].reshape(_NW, 7, 16, _C)
    partials_b = _sc_kernel("b", _make_sc_b, sw, sidx, zeros)

    out, gf = pl.pallas_call(
        _dense_body,
        out_shape=(
            jax.ShapeDtypeStruct((_G, _C), jnp.float32),
            jax.ShapeDtypeStruct((_G, _C), jnp.float32),
        ),
    )(partials_a, partials_b, auxiliary_features, bn_gamma.reshape(1, -1),
      bn_beta.reshape(1, -1), W1, b1.reshape(1, -1), W2, b2.reshape(1, -1))
    return (out, gf)


# speedup vs baseline: 1.5495x; 1.5495x over previous
import functools

import jax
import jax.numpy as jnp
from jax import lax
from jax.experimental import pallas as pl
from jax.experimental.pallas import tpu as pltpu
from jax.experimental.pallas import tpu_sc as plsc

_N = 100000
_G = 1024
_C = 128
_BS = 32
_NB = _N // _BS
_NW = 32
_TRASH = _G

_BPW = 104
_IDPAD = _BPW * (_NW - 1) * _BS + 112 * _BS

_RPW = _BPW * _BS
_NSTEP = 31


def _make_sc_a():
    mesh = plsc.VectorSubcoreMesh(core_axis_name="c", subcore_axis_name="s")

    @functools.partial(
        pl.kernel,
        mesh=mesh,
        out_type=(
            jax.ShapeDtypeStruct((2, _G, _C), jnp.float32),
            jax.ShapeDtypeStruct((_NW, 7, 1, 16), jnp.int32),
        ),
        scratch_types=[
            pltpu.VMEM((3584,), jnp.int32),
            pltpu.VMEM((7, 1, 16), jnp.int32),
            pltpu.VMEM((128,), jnp.int32),
            pltpu.VMEM((128,), jnp.int32),
            pltpu.VMEM((4, _BS, _C), jnp.float32),
            pltpu.VMEM_SHARED((_G + 1, _C), jnp.float32),
        ] + [pltpu.SemaphoreType.DMA] * 8,
    )
    def sc_a(x_hbm, idsf_hbm, zeros_hbm, out_hbm, sidx_hbm,
             ids_f, sidx_v, sidx_f, imp_v, rows_v, accum, *sems):
        cid = lax.axis_index("c")
        sid = lax.axis_index("s")
        wid = sid * 2 + cid
        fsem, ssem = sems[:4], sems[4:]

        b0 = _BPW * wid
        bcnt = jnp.clip(_NB - b0, 0, _BPW)

        pltpu.sync_copy(
            idsf_hbm.at[pl.ds(pl.multiple_of(b0 * _BS, _BS), _BPW * _BS)],
            ids_f.at[pl.ds(0, _BPW * _BS)])

        pltpu.sync_copy(
            zeros_hbm.at[pl.ds(sid * (_G // 16), _G // 16)],
            accum.at[pl.ds(sid * (_G // 16), _G // 16)],
        )

        @pl.when(sid == 0)
        def _():
            pltpu.sync_copy(zeros_hbm.at[pl.ds(0, 1)],
                            accum.at[pl.ds(_G, 1)])

        zvec = jnp.zeros((16,), jnp.int32)

        def cbody(j, m):
            f = ids_f[pl.ds(pl.multiple_of(j * _BS, _BS), 16)][0]
            last = ids_f[pl.ds(pl.multiple_of(j * _BS + 16, 16), 16)][15]
            valid = j < bcnt
            sidx_f[pl.ds(j, 16)] = zvec + jnp.where(
                (f == last) & valid, f, _TRASH)
            imp = ((f != last) & valid).astype(jnp.int32)

            @pl.when(imp == 1)
            def _():
                imp_v[pl.ds(m, 16)] = zvec + j

            return m + imp

        M = lax.fori_loop(0, 112, cbody, wid * 0)

        for t in range(7):
            sidx_v[t, 0, :] = sidx_f[pl.ds(16 * t, 16)]

        pltpu.sync_copy(sidx_v, sidx_hbm.at[wid])
        plsc.subcore_barrier()

        def impure_block(k):
            return imp_v[pl.ds(k, 16)][0]

        def fetch(k, b):
            jg = b0 + impure_block(k)
            return pltpu.async_copy(
                x_hbm.at[pl.ds(pl.multiple_of(jg * _BS, _BS), _BS)],
                rows_v.at[b], fsem[b])

        def wait_fetch(b):
            pltpu.make_async_copy(
                x_hbm.at[pl.ds(0, _BS)], rows_v.at[b], fsem[b]).wait()

        def scat(k, b):
            jl = impure_block(k)
            ia = ids_f[pl.ds(pl.multiple_of(jl * _BS, _BS), 16)]
            ib = ids_f[pl.ds(pl.multiple_of(jl * _BS + 16, 16), 16)]
            pltpu.async_copy(rows_v.at[b, pl.ds(0, 16)], accum.at[ia],
                             ssem[b], add=True)
            pltpu.async_copy(rows_v.at[b, pl.ds(16, 16)], accum.at[ib],
                             ssem[b], add=True)

        def wait_scat(b):
            pltpu.make_async_copy(
                rows_v.at[b], accum.at[pl.ds(0, _BS)], ssem[b]).wait()

        for pb in range(2):
            @pl.when(M > pb)
            def _(pb=pb):
                fetch(pb, pb)

        def ibody(i, carry):
            for b in range(4):
                k = i * 4 + b
                kf = k + 2
                bf = (b + 2) % 4

                @pl.when(kf < M)
                def _():
                    @pl.when(kf >= 4)
                    def _():
                        wait_scat(bf)

                    fetch(kf, bf)

                @pl.when(k < M)
                def _():
                    wait_fetch(b)
                    scat(k, b)

            return carry

        lax.fori_loop(0, (M + 3) // 4, ibody, 0)

        for b in range(4):
            @pl.when(b < jnp.minimum(M, 4))
            def _(b=b):
                wait_scat(b)

        plsc.subcore_barrier()

        pltpu.sync_copy(
            accum.at[pl.ds(sid * (_G // 16), _G // 16)],
            out_hbm.at[cid].at[pl.ds(sid * (_G // 16), _G // 16)],
        )

    return sc_a


def _make_sc_b():
    mesh = plsc.VectorSubcoreMesh(core_axis_name="c", subcore_axis_name="s")

    @functools.partial(
        pl.kernel,
        mesh=mesh,
        out_type=jax.ShapeDtypeStruct((2, _G, _C), jnp.float32),
        scratch_types=[
            pltpu.VMEM((7, 1, 16), jnp.int32),
            pltpu.VMEM((7, 16, _C), jnp.float32),
            pltpu.VMEM_SHARED((_G + 1, _C), jnp.float32),
            pltpu.SemaphoreType.DMA,
        ],
    )
    def sc_b(s_hbm, sidx_hbm, zeros_hbm, out_hbm, sidx_v, srows_v, accum,
             gsem):
        cid = lax.axis_index("c")
        sid = lax.axis_index("s")
        wid = sid * 2 + cid

        pltpu.sync_copy(
            zeros_hbm.at[pl.ds(sid * (_G // 16), _G // 16)],
            accum.at[pl.ds(sid * (_G // 16), _G // 16)],
        )

        @pl.when(sid == 0)
        def _():
            pltpu.sync_copy(zeros_hbm.at[pl.ds(0, 1)],
                            accum.at[pl.ds(_G, 1)])

        pltpu.sync_copy(sidx_hbm.at[wid], sidx_v)

        @pl.when(wid < _NSTEP)
        def _():
            pltpu.sync_copy(s_hbm.at[wid], srows_v)

        plsc.subcore_barrier()

        @pl.when(wid < _NSTEP)
        def _():
            for u in range(7):
                pltpu.sync_copy(srows_v.at[u], accum.at[sidx_v.at[u, 0]],
                                add=True)

        plsc.subcore_barrier()

        pltpu.sync_copy(
            accum.at[pl.ds(sid * (_G // 16), _G // 16)],
            out_hbm.at[cid].at[pl.ds(sid * (_G // 16), _G // 16)],
        )

    return sc_b


_sc_cache = {}


def _sc_kernel(name, maker, *args):
    if name not in _sc_cache:
        _sc_cache[name] = maker()
    return _sc_cache[name](*args)


def _blocksum_body(x_ref, s_ref):
    for i in range(_BPW):
        s_ref[0, i:i + 1, :] = jnp.sum(x_ref[i * _BS:(i + 1) * _BS, :],
                                       axis=0, keepdims=True)


def _blocksum(x):
    s = pl.pallas_call(
        _blocksum_body,
        grid=(_NSTEP,),
        in_specs=[pl.BlockSpec((_RPW, _C), lambda i: (i, 0))],
        out_specs=pl.BlockSpec((1, 112, _C), lambda i: (i, 0, 0)),
        out_shape=jax.ShapeDtypeStruct((_NSTEP, 112, _C), jnp.float32),
    )(x)
    return s.reshape(_NSTEP, 7, 16, _C)


def _dense_body(pa_ref, pb_ref, aux_ref, gam_ref, bet_ref, w1_ref, b1_ref,
                w2_ref, b2_ref, out_ref, gf_ref):
    gf = pa_ref[0] + pa_ref[1] + pb_ref[0] + pb_ref[1]
    gf_ref[...] = gf
    ax = aux_ref[...]

    mg = jnp.mean(gf, axis=0, keepdims=True)
    vg = jnp.mean((gf - mg) ** 2, axis=0, keepdims=True)
    xg = (gf - mg) * lax.rsqrt(vg + 1e-5) * gam_ref[:, :_C] + bet_ref[:, :_C]

    ma = jnp.mean(ax, axis=0, keepdims=True)
    va = jnp.mean((ax - ma) ** 2, axis=0, keepdims=True)
    xa = (ax - ma) * lax.rsqrt(va + 1e-5) * gam_ref[:, _C:] + bet_ref[:, _C:]

    dn = (((1,), (1,)), ((), ()))
    h = lax.dot_general(xg, w1_ref[:, :_C], dn,
                        preferred_element_type=jnp.float32)
    h = h + lax.dot_general(xa, w1_ref[:, _C:], dn,
                            preferred_element_type=jnp.float32)
    h = jnp.maximum(h + b1_ref[...], 0.0)
    out_ref[...] = lax.dot_general(h, w2_ref[...], dn,
                                   preferred_element_type=jnp.float32) + b2_ref[...]


def kernel(raw_node_out, num_graphs, graph_nodes_list, auxiliary_features,
           bn_gamma, bn_beta, W1, b1, W2, b2):
    del num_graphs
    ids_flat = jnp.pad(graph_nodes_list.astype(jnp.int32),
                       (0, _IDPAD - _N))
    zeros = jnp.zeros((_G, _C), jnp.float32)

    partials_a, sidx = _sc_kernel("a", _make_sc_a, raw_node_out, ids_flat,
                                  zeros)
    sw = _blocksum(raw_node_out)
    partials_b = _sc_kernel("b", _make_sc_b, sw, sidx, zeros)

    out, gf = pl.pallas_call(
        _dense_body,
        out_shape=(
            jax.ShapeDtypeStruct((_G, _C), jnp.float32),
            jax.ShapeDtypeStruct((_G, _C), jnp.float32),
        ),
    )(partials_a, partials_b, auxiliary_features, bn_gamma.reshape(1, -1),
      bn_beta.reshape(1, -1), W1, b1.reshape(1, -1), W2, b2.reshape(1, -1))
    return (out, gf)
